# Initial kernel scaffold; baseline (speedup 1.0000x reference)
import jax
import jax.numpy as jnp
from jax.experimental import pallas as pl


def kernel(x, edge_index, W1, b1, W2, b2, W3, b3):
    raise NotImplementedError("write your pallas kernel here")


# speedup vs baseline: 10.6936x; 10.6936x over previous
import functools

import jax
import jax.numpy as jnp
from jax import lax
from jax.experimental import pallas as pl
from jax.experimental.pallas import tpu as pltpu
from jax.experimental.pallas import tpu_sc as plsc

NC = 2
NS = 16
NW = NC * NS
D = 128
BR = 1024


def _mesh():
  return plsc.VectorSubcoreMesh(
      core_axis_name="c", subcore_axis_name="s", num_cores=NC,
      num_subcores=NS)


DW = 16


def _make_deg_kernel(npad, ch):
  rpt = npad // NS

  @functools.partial(
      pl.kernel,
      out_type=jax.ShapeDtypeStruct((NC, npad, DW), jnp.float32),
      mesh=_mesh(),
      scratch_types=[
          pltpu.VMEM((ch, D), jnp.int32),
          pltpu.VMEM((D, DW), jnp.float32),
          pltpu.VMEM_SHARED((npad, DW), jnp.float32),
      ],
  )
  def deg_kernel(dst_hbm, out_hbm, dst_v, buf, shist):
    c = lax.axis_index("c")
    s = lax.axis_index("s")
    wid = s * NC + c

    def fill(j, val):
      buf[j, pl.ds(0, 16)] = jnp.full((16,), val, jnp.float32)
      return val

    lax.fori_loop(0, D, fill, 0.0)
    for k in range(rpt // D):
      pltpu.sync_copy(buf, shist.at[pl.ds(s * rpt + k * D, D)])
    lax.fori_loop(0, D, fill, 1.0)
    plsc.subcore_barrier()

    pltpu.sync_copy(dst_hbm.at[wid], dst_v)

    def body(j, carry):
      pltpu.sync_copy(buf, shist.at[dst_v.at[j]], add=True)
      return carry

    lax.fori_loop(0, ch, body, 0)
    plsc.subcore_barrier()

    pltpu.sync_copy(shist.at[pl.ds(s * rpt, rpt)],
                    out_hbm.at[c, pl.ds(s * rpt, rpt)])

  return deg_kernel


def _make_agg_kernel(npad, ch):
  rpt = npad // NS

  @functools.partial(
      pl.kernel,
      out_type=jax.ShapeDtypeStruct((NC, npad, D), jnp.float32),
      mesh=_mesh(),
      scratch_types=[
          pltpu.VMEM((ch, D), jnp.int32),
          pltpu.VMEM((ch, D), jnp.int32),
          pltpu.VMEM((D, D), jnp.float32),
          pltpu.VMEM_SHARED((npad, D), jnp.float32),
      ],
  )
  def agg_kernel(g_hbm, src_hbm, dst_hbm, out_hbm, src_v, dst_v, rowbuf, acc):
    c = lax.axis_index("c")
    s = lax.axis_index("s")
    wid = s * NC + c

    def zrow(j, carry):
      for k in range(D // 16):
        rowbuf[j, pl.ds(k * 16, 16)] = jnp.zeros((16,), jnp.float32)
      return carry

    lax.fori_loop(0, D, zrow, 0)
    for k in range(rpt // D):
      pltpu.sync_copy(rowbuf, acc.at[pl.ds(s * rpt + k * D, D)])
    plsc.subcore_barrier()

    pltpu.sync_copy(src_hbm.at[wid], src_v)
    pltpu.sync_copy(dst_hbm.at[wid], dst_v)

    def body(j, carry):
      pltpu.sync_copy(g_hbm.at[src_v.at[j]], rowbuf)
      pltpu.sync_copy(rowbuf, acc.at[dst_v.at[j]], add=True)
      return carry

    lax.fori_loop(0, ch, body, 0)
    plsc.subcore_barrier()

    pltpu.sync_copy(acc.at[pl.ds(s * rpt, rpt)],
                    out_hbm.at[c, pl.ds(s * rpt, rpt)])

  return agg_kernel


def _tc_pre(xp, W1, degp):
  npad = xp.shape[0]

  def body(x_ref, w_ref, dp_ref, g_ref, dinv_ref):
    deg = dp_ref[0, :, 0:1] + dp_ref[1, :, 0:1] + 1.0
    dinv = 1.0 / jnp.sqrt(deg)
    dinv_ref[...] = dinv
    g_ref[...] = jnp.dot(
        x_ref[...], w_ref[...], preferred_element_type=jnp.float32) * dinv

  return pl.pallas_call(
      body,
      grid=(npad // BR,),
      in_specs=[
          pl.BlockSpec((BR, D), lambda i: (i, 0)),
          pl.BlockSpec((D, D), lambda i: (0, 0)),
          pl.BlockSpec((NC, BR, DW), lambda i: (0, i, 0)),
      ],
      out_specs=[
          pl.BlockSpec((BR, D), lambda i: (i, 0)),
          pl.BlockSpec((BR, 1), lambda i: (i, 0)),
      ],
      out_shape=[
          jax.ShapeDtypeStruct((npad, D), jnp.float32),
          jax.ShapeDtypeStruct((npad, 1), jnp.float32),
      ],
  )(xp, W1, degp)


def _tc_mid(p, g, dinv, b, W):
  npad = g.shape[0]

  def body(p_ref, g_ref, dinv_ref, b_ref, w_ref, out_ref):
    t = (p_ref[0] + p_ref[1] + g_ref[...]) * dinv_ref[...] + b_ref[...]
    t = jnp.maximum(t, 0.0)
    out_ref[...] = jnp.dot(
        t, w_ref[...], preferred_element_type=jnp.float32) * dinv_ref[...]

  return pl.pallas_call(
      body,
      grid=(npad // BR,),
      in_specs=[
          pl.BlockSpec((NC, BR, D), lambda i: (0, i, 0)),
          pl.BlockSpec((BR, D), lambda i: (i, 0)),
          pl.BlockSpec((BR, 1), lambda i: (i, 0)),
          pl.BlockSpec((1, D), lambda i: (0, 0)),
          pl.BlockSpec((D, D), lambda i: (0, 0)),
      ],
      out_specs=pl.BlockSpec((BR, D), lambda i: (i, 0)),
      out_shape=jax.ShapeDtypeStruct((npad, D), jnp.float32),
  )(p, g, dinv, b, W)


def _tc_post(p, g, dinv, b):
  npad = g.shape[0]

  def body(p_ref, g_ref, dinv_ref, b_ref, out_ref):
    out_ref[...] = (
        (p_ref[0] + p_ref[1] + g_ref[...]) * dinv_ref[...] + b_ref[...])

  return pl.pallas_call(
      body,
      grid=(npad // BR,),
      in_specs=[
          pl.BlockSpec((NC, BR, D), lambda i: (0, i, 0)),
          pl.BlockSpec((BR, D), lambda i: (i, 0)),
          pl.BlockSpec((BR, 1), lambda i: (i, 0)),
          pl.BlockSpec((1, D), lambda i: (0, 0)),
      ],
      out_specs=pl.BlockSpec((BR, D), lambda i: (i, 0)),
      out_shape=jax.ShapeDtypeStruct((npad, D), jnp.float32),
  )(p, g, dinv, b)


def kernel(x, edge_index, W1, b1, W2, b2, W3, b3):
  N, d_in = x.shape
  E = edge_index.shape[1]

  npad = -(-(N + 1) // (NS * D)) * (NS * D)
  ch = -(-E // (NW * D))
  epad = NW * ch * D

  src = edge_index[0].astype(jnp.int32)
  dst = edge_index[1].astype(jnp.int32)
  pad = jnp.full((epad - E,), N, jnp.int32)
  src3 = jnp.concatenate([src, pad]).reshape(NW, ch, D)
  dst3 = jnp.concatenate([dst, pad]).reshape(NW, ch, D)
  xp = jnp.concatenate([x, jnp.zeros((npad - N, d_in), x.dtype)])

  degp = _make_deg_kernel(npad, ch)(dst3)

  agg = _make_agg_kernel(npad, ch)

  g, dinv = _tc_pre(xp, W1, degp)
  p = agg(g, src3, dst3)
  g = _tc_mid(p, g, dinv, b1.reshape(1, D), W2)
  p = agg(g, src3, dst3)
  g = _tc_mid(p, g, dinv, b2.reshape(1, D), W3)
  p = agg(g, src3, dst3)
  out = _tc_post(p, g, dinv, b3.reshape(1, D))
  return out[:N]
